# baseline (speedup 1.0000x reference)
import functools

import jax
import jax.numpy as jnp
from jax import lax
from jax.experimental import pallas as pl
from jax.experimental.pallas import tpu as pltpu
from jax.experimental.pallas import tpu_sc as plsc

T = 200
D = 32
NC = 2
NS = 16
NW = NC * NS
BPW = 128
NBUF = 8
LEAD = 6
TB = 4
XCOL = 40


CH = 512
NCH_FULL = 1953
TAIL = 64


def _tbody(tokt_hbm, out_hbm, ibuf, obuf, *sems):
    sem_i = sems[:2]
    sem_o = sems[2:4]
    wid = lax.axis_index("s") * NC + lax.axis_index("c")

    lanes = lax.broadcasted_iota(jnp.int32, (16,), 0)
    dv0 = lanes
    dv1 = lanes + 16

    def issue_in(c, p):
        pltpu.async_copy(tokt_hbm.at[:, pl.ds(c * CH, CH)],
                         ibuf.at[p, :, pl.ds(0, CH)], sem_i[p])

    def wait_in(p):
        pltpu.make_async_copy(tokt_hbm.at[:, pl.ds(0, CH)],
                              ibuf.at[p, :, pl.ds(0, CH)], sem_i[p]).wait()

    def issue_out(c, p):
        pltpu.async_copy(obuf.at[p], out_hbm.at[pl.ds(c * (CH // 4), CH // 4), :],
                         sem_o[p])

    def wait_out(p):
        pltpu.make_async_copy(obuf.at[p], out_hbm.at[pl.ds(0, CH // 4), :],
                              sem_o[p]).wait()

    def transform(p, width):
        ib = ibuf.at[p]

        def row_body(v, carry):
            for j in range(4):
                tv = jnp.broadcast_to(4 * v + j, (16,)).astype(jnp.int32)
                lo = plsc.load_gather(ib, [dv0, tv])
                hi = plsc.load_gather(ib, [dv1, tv])
                obuf[p, v, pl.ds(j * 32, 16)] = lo
                obuf[p, v, pl.ds(j * 32 + 16, 16)] = hi
            return carry

        lax.fori_loop(0, width // 4, row_body, 0, unroll=2)

    @pl.when(wid < NCH_FULL)
    def _():
        issue_in(wid, 0)

    def ch_body(i, carry):
        for p in range(2):
            c = wid + (2 * i + p) * NW
            c_next = c + NW

            @pl.when(c_next < NCH_FULL)
            def _():
                issue_in(c_next, 1 - p)

            @pl.when(c < NCH_FULL)
            def _():
                wait_in(p)

                @pl.when(c >= 2 * NW)
                def _():
                    wait_out(p)

                transform(p, CH)
                issue_out(c, p)
        return carry

    lax.fori_loop(0, (NCH_FULL + 2 * NW - 1) // (2 * NW), ch_body, 0)

    for p in range(2):
        @pl.when(wid + p * NW < NCH_FULL)
        def _():
            wait_out(p)

    @pl.when(wid == 0)
    def _():
        pltpu.sync_copy(tokt_hbm.at[:, pl.ds(1000000 - 128, 128)],
                        ibuf.at[0, :, pl.ds(0, 128)])
        transform(0, 128)
        pltpu.sync_copy(obuf.at[0, pl.ds(0, 32), :],
                        out_hbm.at[pl.ds(250000 - 32, 32), :])


def _body(x_hbm, tok_hbm, pos_hbm, out_hbm, idxt, xtmp, pos_v, grows, stg, *sems):
    sem_g = sems[:NBUF]
    sem_w = sems[NBUF:NBUF + 2]
    wid = lax.axis_index("s") * NC + lax.axis_index("c")
    base = wid * BPW

    pltpu.sync_copy(pos_hbm, pos_v)

    lanes = lax.broadcasted_iota(jnp.int32, (16,), 0)
    lanes_hi = lanes + 16

    for s in range(T // XCOL):
        pltpu.sync_copy(
            x_hbm.at[pl.ds(base, BPW), pl.ds(s * XCOL, XCOL)], xtmp)

        def xt_body(p, carry):
            b0 = (p % (BPW // 16)) * 16
            tl = p // (BPW // 16)
            tv = jnp.broadcast_to(tl, (16,)).astype(jnp.int32)
            src = plsc.load_gather(xtmp, [b0 + lanes, tv])
            idxt[s * XCOL + tl, pl.ds(b0, 16)] = src
            return carry

        lax.fori_loop(0, (BPW // 16) * XCOL, xt_body, 0, unroll=4)

    def issue_gather(t, g):
        pltpu.async_copy(tok_hbm.at[idxt.at[t, pl.ds(0, BPW)]], grows.at[g], sem_g[g])

    def wait_gather(g):
        pltpu.make_async_copy(
            tok_hbm.at[pl.ds(0, BPW)], grows.at[g], sem_g[g]).wait()

    def issue_wb(t0, s):
        for tl in range(TB):
            for dr in range(4):
                pltpu.async_copy(
                    stg.at[s, tl, pl.ds(dr * 8, 8), pl.ds(0, BPW)],
                    out_hbm.at[t0 + tl].at[dr].at[wid], sem_w[s])

    def wait_wb(s):
        for _ in range(TB * 4):
            pltpu.make_async_copy(
                stg.at[s, 0, pl.ds(0, 8), pl.ds(0, BPW)],
                out_hbm.at[0].at[0].at[0], sem_w[s]).wait()

    for g in range(LEAD):
        issue_gather(g, g)

    def slot(t, g, k, s):
        t_pf = t + LEAD

        @pl.when(t_pf < T)
        def _():
            issue_gather(t_pf, (g + LEAD) % NBUF)

        wait_gather(g)
        p0 = pos_v[t, pl.ds(0, 16)]
        p1 = pos_v[t, pl.ds(16, 16)]

        stg2 = stg.at[s, k]

        def add_body(b, carry):
            bv = jnp.broadcast_to(b, (16,)).astype(jnp.int32)
            v0 = grows[g, b, pl.ds(0, 16)] + p0
            v1 = grows[g, b, pl.ds(16, 16)] + p1
            plsc.store_scatter(stg2, [lanes, bv], v0)
            plsc.store_scatter(stg2, [lanes_hi, bv], v1)
            return carry

        lax.fori_loop(0, BPW, add_body, 0, unroll=8)

    def grp_body(i, carry):
        t0 = i * (2 * TB)
        for half in range(2):
            tb0 = t0 + half * TB

            @pl.when(i > 0)
            def _():
                wait_wb(half)

            for k in range(TB):
                slot(tb0 + k, half * TB + k, k, half)
            issue_wb(tb0, half)
        return carry

    lax.fori_loop(0, T // (2 * TB), grp_body, 0)

    for s in range(2):
        wait_wb(s)


def kernel(x, token_table, pos_table):
    batch, maxlen = x.shape

    mesh = plsc.VectorSubcoreMesh(core_axis_name="c", subcore_axis_name="s")
    tfn = pl.kernel(
        _tbody,
        out_type=jax.ShapeDtypeStruct((250000, 128), jnp.float32),
        mesh=mesh,
        scratch_types=[
            pltpu.VMEM((2, D, CH + 1), jnp.float32),
            pltpu.VMEM((2, CH // 4, 128), jnp.float32),
        ] + [pltpu.SemaphoreType.DMA] * 4,
        compiler_params=pltpu.CompilerParams(
            use_tc_tiling_on_sc=False, needs_layout_passes=False),
    )
    tok_lin = tfn(token_table.T).reshape(1000000, D)

    fn = pl.kernel(
        _body,
        out_type=jax.ShapeDtypeStruct((T, 4, NW, 8, BPW), jnp.float32),
        mesh=mesh,
        scratch_types=[
            pltpu.VMEM((T, BPW), jnp.int32),
            pltpu.VMEM((BPW, XCOL), jnp.int32),
            pltpu.VMEM((T, D), jnp.float32),
            pltpu.VMEM((NBUF, BPW, D), jnp.float32),
            pltpu.VMEM((2, TB, D, 129), jnp.float32),
        ] + [pltpu.SemaphoreType.DMA] * (NBUF + 2),
        compiler_params=pltpu.CompilerParams(
            use_tc_tiling_on_sc=False, needs_layout_passes=False),
    )
    o5 = fn(x.astype(jnp.int32), tok_lin, pos_table)
    return o5.transpose(2, 4, 0, 1, 3).reshape(batch, maxlen, D)


# speedup vs baseline: 4.2802x; 4.2802x over previous
import functools

import jax
import jax.numpy as jnp
from jax import lax
from jax.experimental import pallas as pl
from jax.experimental.pallas import tpu as pltpu
from jax.experimental.pallas import tpu_sc as plsc

T = 200
D = 32
NC = 2
NS = 16
NW = NC * NS
BPW = 128
NBUF = 8
LEAD = 6
TB = 4
XCOL = 40


def _body(x_hbm, tok_hbm, pos_hbm, out_hbm, idxt, xtmp, pos_v, grows, stg, *sems):
    sem_g = sems[:NBUF]
    sem_w = sems[NBUF:NBUF + 2]
    wid = lax.axis_index("s") * NC + lax.axis_index("c")
    base = wid * BPW

    pltpu.sync_copy(pos_hbm, pos_v)

    lanes = lax.broadcasted_iota(jnp.int32, (16,), 0)
    lanes_hi = lanes + 16

    for s in range(T // XCOL):
        pltpu.sync_copy(
            x_hbm.at[pl.ds(base, BPW), pl.ds(s * XCOL, XCOL)], xtmp)

        def xt_body(p, carry):
            b0 = (p % (BPW // 16)) * 16
            tl = p // (BPW // 16)
            tv = jnp.broadcast_to(tl, (16,)).astype(jnp.int32)
            src = plsc.load_gather(xtmp, [b0 + lanes, tv])
            idxt[s * XCOL + tl, pl.ds(b0, 16)] = src
            return carry

        lax.fori_loop(0, (BPW // 16) * XCOL, xt_body, 0, unroll=4)

    def issue_gather(t, g):
        pltpu.async_copy(tok_hbm.at[idxt.at[t, pl.ds(0, BPW)]], grows.at[g], sem_g[g])

    def wait_gather(g):
        pltpu.make_async_copy(
            tok_hbm.at[pl.ds(0, BPW)], grows.at[g], sem_g[g]).wait()

    def issue_wb(t0, s):
        for tl in range(TB):
            for dr in range(4):
                pltpu.async_copy(
                    stg.at[s, tl, pl.ds(dr * 8, 8), pl.ds(0, BPW)],
                    out_hbm.at[t0 + tl].at[dr].at[wid], sem_w[s])

    def wait_wb(s):
        for _ in range(TB * 4):
            pltpu.make_async_copy(
                stg.at[s, 0, pl.ds(0, 8), pl.ds(0, BPW)],
                out_hbm.at[0].at[0].at[0], sem_w[s]).wait()

    for g in range(LEAD):
        issue_gather(g, g)

    def slot(t, g, k, s):
        t_pf = t + LEAD

        @pl.when(t_pf < T)
        def _():
            issue_gather(t_pf, (g + LEAD) % NBUF)

        wait_gather(g)
        p0 = pos_v[t, pl.ds(0, 16)]
        p1 = pos_v[t, pl.ds(16, 16)]

        stg2 = stg.at[s, k]

        def add_body(b, carry):
            bv = jnp.broadcast_to(b, (16,)).astype(jnp.int32)
            v0 = grows[g, b, pl.ds(0, 16)] + p0
            v1 = grows[g, b, pl.ds(16, 16)] + p1
            plsc.store_scatter(stg2, [lanes, bv], v0)
            plsc.store_scatter(stg2, [lanes_hi, bv], v1)
            return carry

        lax.fori_loop(0, BPW, add_body, 0, unroll=8)

    def grp_body(i, carry):
        t0 = i * (2 * TB)
        for half in range(2):
            tb0 = t0 + half * TB

            @pl.when(i > 0)
            def _():
                wait_wb(half)

            for k in range(TB):
                slot(tb0 + k, half * TB + k, k, half)
            issue_wb(tb0, half)
        return carry

    lax.fori_loop(0, T // (2 * TB), grp_body, 0)

    for s in range(2):
        wait_wb(s)


def kernel(x, token_table, pos_table):
    batch, maxlen = x.shape

    mesh = plsc.VectorSubcoreMesh(core_axis_name="c", subcore_axis_name="s")
    fn = pl.kernel(
        _body,
        out_type=jax.ShapeDtypeStruct((T, 4, NW, 8, BPW), jnp.float32),
        mesh=mesh,
        scratch_types=[
            pltpu.VMEM((T, BPW), jnp.int32),
            pltpu.VMEM((BPW, XCOL), jnp.int32),
            pltpu.VMEM((T, D), jnp.float32),
            pltpu.VMEM((NBUF, BPW, D), jnp.float32),
            pltpu.VMEM((2, TB, D, 129), jnp.float32),
        ] + [pltpu.SemaphoreType.DMA] * (NBUF + 2),
        compiler_params=pltpu.CompilerParams(
            use_tc_tiling_on_sc=False, needs_layout_passes=False),
    )
    o5 = fn(x.astype(jnp.int32), token_table, pos_table)
    return o5.transpose(2, 4, 0, 1, 3).reshape(batch, maxlen, D)
